# baseline (speedup 1.0000x reference)
import functools

import jax
import jax.numpy as jnp
from jax import lax
from jax.experimental import pallas as pl
from jax.experimental.pallas import tpu as pltpu
from jax.experimental.pallas import tpu_sc as plsc

EMB = 128
BATCH = 4096
NEG = 20
L = 16
NC, NS = 2, 16
NW = NC * NS
BPW = BATCH // NW
GROUPS = BPW // L
NROWS = L * NEG
NCH = 4
CHUNK = NROWS // NCH


def _sc_scores(ctr_ids, ctx_ids, neg_flat, u_global, v_global, u_reg, v_reg):
    mesh = plsc.VectorSubcoreMesh(
        core_axis_name="c", subcore_axis_name="s", num_cores=NC, num_subcores=NS
    )

    @functools.partial(
        pl.kernel,
        out_type=jax.ShapeDtypeStruct((1 + NEG, BATCH), jnp.float32),
        mesh=mesh,
        compiler_params=pltpu.CompilerParams(needs_layout_passes=False),
        scratch_types=[
            pltpu.VMEM((BPW,), jnp.int32),
            pltpu.VMEM((BPW,), jnp.int32),
            pltpu.VMEM((BPW * NEG,), jnp.int32),
            pltpu.VMEM((L, EMB), jnp.float32),
            pltpu.VMEM((L, EMB), jnp.float32),
            pltpu.VMEM((L, EMB), jnp.float32),
            pltpu.VMEM((L, EMB), jnp.float32),
            pltpu.VMEM((NROWS, EMB), jnp.float32),
            pltpu.VMEM((NROWS, EMB), jnp.float32),
            pltpu.VMEM((1 + NEG, BPW), jnp.float32),
            pltpu.SemaphoreType.DMA,
        ],
    )
    def body(ctr_hbm, ctx_hbm, neg_hbm, ug_hbm, vg_hbm, ur_hbm, vr_hbm, out_hbm,
             ctr_idx, ctx_idx, neg_idx, ctr_g, ctr_r, ctx_g, ctx_r,
             neg_g, neg_r, scores, sem):
        wid = lax.axis_index("s") * NC + lax.axis_index("c")
        base = pl.multiple_of(wid * BPW, BPW)
        nbase = pl.multiple_of(wid * (BPW * NEG), BPW * NEG)
        pltpu.sync_copy(ctr_hbm.at[pl.ds(base, BPW)], ctr_idx)
        pltpu.sync_copy(ctx_hbm.at[pl.ds(base, BPW)], ctx_idx)
        pltpu.sync_copy(neg_hbm.at[pl.ds(nbase, BPW * NEG)], neg_idx)

        lanes = lax.iota(jnp.int32, L)
        klanes = lanes * NEG

        for g in range(GROUPS):
            handles = [
                pltpu.async_copy(ug_hbm.at[ctr_idx.at[pl.ds(g * L, L)]], ctr_g, sem),
                pltpu.async_copy(ur_hbm.at[ctr_idx.at[pl.ds(g * L, L)]], ctr_r, sem),
                pltpu.async_copy(vg_hbm.at[ctx_idx.at[pl.ds(g * L, L)]], ctx_g, sem),
                pltpu.async_copy(vr_hbm.at[ctx_idx.at[pl.ds(g * L, L)]], ctx_r, sem),
            ]
            for c in range(NCH):
                off = g * NROWS + c * CHUNK
                dst = pl.ds(c * CHUNK, CHUNK)
                handles.append(pltpu.async_copy(
                    vg_hbm.at[neg_idx.at[pl.ds(off, CHUNK)]], neg_g.at[dst], sem))
                handles.append(pltpu.async_copy(
                    vr_hbm.at[neg_idx.at[pl.ds(off, CHUNK)]], neg_r.at[dst], sem))
            for h in handles:
                h.wait()

            def dbody(d, accs):
                dcol = jnp.full((L,), d, jnp.int32)
                c = (plsc.load_gather(ctr_g, [lanes, dcol])
                     + plsc.load_gather(ctr_r, [lanes, dcol]))
                x = (plsc.load_gather(ctx_g, [lanes, dcol])
                     + plsc.load_gather(ctx_r, [lanes, dcol]))
                new = [accs[0] + x * c]
                for k in range(NEG):
                    rows = klanes + k
                    n = (plsc.load_gather(neg_g, [rows, dcol])
                         + plsc.load_gather(neg_r, [rows, dcol]))
                    new.append(accs[k + 1] + n * c)
                return tuple(new)

            accs = lax.fori_loop(
                0, EMB, dbody,
                tuple(jnp.zeros((L,), jnp.float32) for _ in range(1 + NEG)))
            for k in range(1 + NEG):
                scores[k, pl.ds(g * L, L)] = accs[k]

        pltpu.sync_copy(scores, out_hbm.at[:, pl.ds(base, BPW)])

    return body(ctr_ids, ctx_ids, neg_flat, u_global, v_global, u_reg, v_reg)


def _tc_loss(scores):
    def body(s_ref, o_ref):
        s = s_ref[...]

        def logsig(x):
            return jnp.minimum(x, 0.0) - jnp.log1p(jnp.exp(-jnp.abs(x)))

        p = logsig(jnp.clip(s[0:1, :], -10.0, 10.0))
        n = logsig(-jnp.clip(s[1:1 + NEG, :], -10.0, 10.0))
        o_ref[0, 0] = -(jnp.sum(p) + jnp.sum(n)) / BATCH

    return pl.pallas_call(
        body,
        out_shape=jax.ShapeDtypeStruct((1, 1), jnp.float32),
        out_specs=pl.BlockSpec(memory_space=pltpu.SMEM),
    )(scores)


def kernel(center_ids, context_ids, neg_ids, u_global, v_global, u_reg, v_reg):
    ctr = center_ids.astype(jnp.int32)
    ctx = context_ids.astype(jnp.int32)
    neg = neg_ids.astype(jnp.int32).reshape(-1)
    scores = _sc_scores(ctr, ctx, neg, u_global, v_global, u_reg, v_reg)
    return _tc_loss(scores)[0, 0]


# speedup vs baseline: 3.7148x; 3.7148x over previous
import functools

import jax
import jax.numpy as jnp
from jax import lax
from jax.experimental import pallas as pl
from jax.experimental.pallas import tpu as pltpu
from jax.experimental.pallas import tpu_sc as plsc

EMB = 128
BATCH = 4096
NEG = 20
L = 16
NC, NS = 2, 16
NW = NC * NS
BPW = BATCH // NW
GROUPS = BPW // L
NROWS = L * NEG
NCH = 4
CHUNK = NROWS // NCH


def _sc_scores(ctr_ids, ctx_ids, neg_flat, u_global, v_global, u_reg, v_reg):
    mesh = plsc.VectorSubcoreMesh(
        core_axis_name="c", subcore_axis_name="s", num_cores=NC, num_subcores=NS
    )

    @functools.partial(
        pl.kernel,
        out_type=jax.ShapeDtypeStruct((1 + NEG, BATCH), jnp.float32),
        mesh=mesh,
        compiler_params=pltpu.CompilerParams(needs_layout_passes=False),
        scratch_types=[
            pltpu.VMEM((BPW,), jnp.int32),
            pltpu.VMEM((BPW,), jnp.int32),
            pltpu.VMEM((BPW * NEG,), jnp.int32),
            pltpu.VMEM((L, EMB), jnp.float32),
            pltpu.VMEM((L, EMB), jnp.float32),
            pltpu.VMEM((L, EMB), jnp.float32),
            pltpu.VMEM((L, EMB), jnp.float32),
            pltpu.VMEM((NROWS, EMB), jnp.float32),
            pltpu.VMEM((NROWS, EMB), jnp.float32),
            pltpu.VMEM((1 + NEG, BPW), jnp.float32),
            pltpu.SemaphoreType.DMA,
        ],
    )
    def body(ctr_hbm, ctx_hbm, neg_hbm, ug_hbm, vg_hbm, ur_hbm, vr_hbm, out_hbm,
             ctr_idx, ctx_idx, neg_idx, ctr_g, ctr_r, ctx_g, ctx_r,
             neg_g, neg_r, scores, sem):
        wid = lax.axis_index("s") * NC + lax.axis_index("c")
        base = pl.multiple_of(wid * BPW, BPW)
        nbase = pl.multiple_of(wid * (BPW * NEG), BPW * NEG)
        pltpu.sync_copy(ctr_hbm.at[pl.ds(base, BPW)], ctr_idx)
        pltpu.sync_copy(ctx_hbm.at[pl.ds(base, BPW)], ctx_idx)
        pltpu.sync_copy(neg_hbm.at[pl.ds(nbase, BPW * NEG)], neg_idx)

        lanes = lax.iota(jnp.int32, L)
        last_lane = lanes == (L - 1)

        for g in range(GROUPS):
            handles = [
                pltpu.async_copy(ug_hbm.at[ctr_idx.at[pl.ds(g * L, L)]], ctr_g, sem),
                pltpu.async_copy(ur_hbm.at[ctr_idx.at[pl.ds(g * L, L)]], ctr_r, sem),
                pltpu.async_copy(vg_hbm.at[ctx_idx.at[pl.ds(g * L, L)]], ctx_g, sem),
                pltpu.async_copy(vr_hbm.at[ctx_idx.at[pl.ds(g * L, L)]], ctx_r, sem),
            ]
            for c in range(NCH):
                off = g * NROWS + c * CHUNK
                dst = pl.ds(c * CHUNK, CHUNK)
                handles.append(pltpu.async_copy(
                    vg_hbm.at[neg_idx.at[pl.ds(off, CHUNK)]], neg_g.at[dst], sem))
                handles.append(pltpu.async_copy(
                    vr_hbm.at[neg_idx.at[pl.ds(off, CHUNK)]], neg_r.at[dst], sem))
            for h in handles:
                h.wait()

            def bbody(b, carry):
                c = [ctr_g[b, pl.ds(16 * j, 16)] + ctr_r[b, pl.ds(16 * j, 16)]
                     for j in range(EMB // L)]
                col = jnp.full((L,), g * L, jnp.int32) + b

                def emit(k, acc):
                    tot = plsc.cumsum(acc)
                    plsc.store_scatter(
                        scores, [jnp.full((L,), k, jnp.int32), col], tot,
                        mask=last_lane)

                acc = (ctx_g[b, pl.ds(0, 16)] + ctx_r[b, pl.ds(0, 16)]) * c[0]
                for j in range(1, EMB // L):
                    acc += (ctx_g[b, pl.ds(16 * j, 16)]
                            + ctx_r[b, pl.ds(16 * j, 16)]) * c[j]
                emit(0, acc)
                for k in range(NEG):
                    row = b * NEG + k
                    acc = (neg_g[row, pl.ds(0, 16)]
                           + neg_r[row, pl.ds(0, 16)]) * c[0]
                    for j in range(1, EMB // L):
                        acc += (neg_g[row, pl.ds(16 * j, 16)]
                                + neg_r[row, pl.ds(16 * j, 16)]) * c[j]
                    emit(k + 1, acc)
                return carry

            lax.fori_loop(0, L, bbody, 0)

        pltpu.sync_copy(scores, out_hbm.at[:, pl.ds(base, BPW)])

    return body(ctr_ids, ctx_ids, neg_flat, u_global, v_global, u_reg, v_reg)


def _tc_loss(scores):
    def body(s_ref, o_ref):
        s = s_ref[...]

        def logsig(x):
            return jnp.minimum(x, 0.0) - jnp.log1p(jnp.exp(-jnp.abs(x)))

        p = logsig(jnp.clip(s[0:1, :], -10.0, 10.0))
        n = logsig(-jnp.clip(s[1:1 + NEG, :], -10.0, 10.0))
        o_ref[0, 0] = -(jnp.sum(p) + jnp.sum(n)) / BATCH

    return pl.pallas_call(
        body,
        out_shape=jax.ShapeDtypeStruct((1, 1), jnp.float32),
        out_specs=pl.BlockSpec(memory_space=pltpu.SMEM),
    )(scores)


def kernel(center_ids, context_ids, neg_ids, u_global, v_global, u_reg, v_reg):
    ctr = center_ids.astype(jnp.int32)
    ctx = context_ids.astype(jnp.int32)
    neg = neg_ids.astype(jnp.int32).reshape(-1)
    scores = _sc_scores(ctr, ctx, neg, u_global, v_global, u_reg, v_reg)
    return _tc_loss(scores)[0, 0]


# speedup vs baseline: 4.8592x; 1.3081x over previous
import functools

import jax
import jax.numpy as jnp
from jax import lax
from jax.experimental import pallas as pl
from jax.experimental.pallas import tpu as pltpu
from jax.experimental.pallas import tpu_sc as plsc

EMB = 128
BATCH = 4096
NEG = 20
L = 16
NV = EMB // L
NC, NS = 2, 16
NW = NC * NS
BPW = BATCH // NW
BB = 8
NT = BPW // BB
NROWS = BB * NEG
CHUNK = 80
NCH = NROWS // CHUNK


def _sc_scores(ctr_ids, ctx_ids, neg_flat, u_global, v_global, u_reg, v_reg):
    mesh = plsc.VectorSubcoreMesh(
        core_axis_name="c", subcore_axis_name="s", num_cores=NC, num_subcores=NS
    )

    row_buf = lambda n: pltpu.VMEM((n, EMB), jnp.float32)
    set_bufs = [row_buf(BB)] * 4 + [row_buf(NROWS)] * 2

    @functools.partial(
        pl.kernel,
        out_type=jax.ShapeDtypeStruct((1 + NEG, BATCH), jnp.float32),
        mesh=mesh,
        compiler_params=pltpu.CompilerParams(needs_layout_passes=False),
        scratch_types=[
            pltpu.VMEM((BPW,), jnp.int32),
            pltpu.VMEM((BPW,), jnp.int32),
            pltpu.VMEM((BPW * NEG,), jnp.int32),
            *set_bufs,
            *set_bufs,
            pltpu.VMEM((1 + NEG, BPW), jnp.float32),
            pltpu.SemaphoreType.DMA,
            pltpu.SemaphoreType.DMA,
        ],
    )
    def body(ctr_hbm, ctx_hbm, neg_hbm, ug_hbm, vg_hbm, ur_hbm, vr_hbm, out_hbm,
             ctr_idx, ctx_idx, neg_idx, *rest):
        bufs = (rest[0:6], rest[6:12])
        scores = rest[12]
        sems = (rest[13], rest[14])
        wid = lax.axis_index("s") * NC + lax.axis_index("c")
        base = pl.multiple_of(wid * BPW, BPW)
        nbase = pl.multiple_of(wid * (BPW * NEG), BPW * NEG)
        pltpu.sync_copy(ctr_hbm.at[pl.ds(base, BPW)], ctr_idx)
        pltpu.sync_copy(ctx_hbm.at[pl.ds(base, BPW)], ctx_idx)
        pltpu.sync_copy(neg_hbm.at[pl.ds(nbase, BPW * NEG)], neg_idx)

        lanes = lax.iota(jnp.int32, L)
        lane0 = lanes == 0
        perms = [lanes ^ sh for sh in (8, 4, 2, 1)]

        def transfers(t, bset):
            ctr_g, ctr_r, ctx_g, ctx_r, neg_g, neg_r = bset
            toff = pl.multiple_of(t * BB, BB)
            pairs = [
                (ug_hbm.at[ctr_idx.at[pl.ds(toff, BB)]], ctr_g),
                (ur_hbm.at[ctr_idx.at[pl.ds(toff, BB)]], ctr_r),
                (vg_hbm.at[ctx_idx.at[pl.ds(toff, BB)]], ctx_g),
                (vr_hbm.at[ctx_idx.at[pl.ds(toff, BB)]], ctx_r),
            ]
            noff = pl.multiple_of(t * NROWS, CHUNK)
            for c in range(NCH):
                src = neg_idx.at[pl.ds(noff + c * CHUNK, CHUNK)]
                dst = pl.ds(c * CHUNK, CHUNK)
                pairs.append((vg_hbm.at[src], neg_g.at[dst]))
                pairs.append((vr_hbm.at[src], neg_r.at[dst]))
            return pairs

        def issue(t, s):
            for src, dst in transfers(t, bufs[s]):
                pltpu.async_copy(src, dst, sems[s])

        def drain(t, s):
            for src, dst in transfers(t, bufs[s]):
                pltpu.make_async_copy(src, dst, sems[s]).wait()

        shuffle_dn = lax.GatherDimensionNumbers(
            offset_dims=(), collapsed_slice_dims=(0,), start_index_map=(0,))

        def reduce_full(acc):
            for p in perms:
                acc = acc + lax.gather(
                    acc, p[:, None], shuffle_dn, slice_sizes=(1,),
                    mode=lax.GatherScatterMode.PROMISE_IN_BOUNDS)
            return acc

        def compute(t, s):
            ctr_g, ctr_r, ctx_g, ctx_r, neg_g, neg_r = bufs[s]

            def bbody(b, carry):
                c = [ctr_g[b, pl.ds(16 * j, 16)] + ctr_r[b, pl.ds(16 * j, 16)]
                     for j in range(NV)]
                col = jnp.full((L,), t * BB, jnp.int32) + b

                def emit(k, acc):
                    plsc.store_scatter(
                        scores, [jnp.full((L,), k, jnp.int32), col],
                        reduce_full(acc), mask=lane0)

                acc = (ctx_g[b, pl.ds(0, 16)] + ctx_r[b, pl.ds(0, 16)]) * c[0]
                for j in range(1, NV):
                    acc += (ctx_g[b, pl.ds(16 * j, 16)]
                            + ctx_r[b, pl.ds(16 * j, 16)]) * c[j]
                emit(0, acc)
                for k in range(NEG):
                    row = b * NEG + k
                    acc = (neg_g[row, pl.ds(0, 16)]
                           + neg_r[row, pl.ds(0, 16)]) * c[0]
                    for j in range(1, NV):
                        acc += (neg_g[row, pl.ds(16 * j, 16)]
                                + neg_r[row, pl.ds(16 * j, 16)]) * c[j]
                    emit(k + 1, acc)
                return carry

            lax.fori_loop(0, BB, bbody, 0)

        issue(0, 0)
        issue(1, 1)

        def tbody(tt, carry):
            t = tt * 2
            drain(t, 0)
            compute(t, 0)

            @pl.when(t + 2 < NT)
            def _():
                issue(t + 2, 0)

            drain(t + 1, 1)
            compute(t + 1, 1)

            @pl.when(t + 3 < NT)
            def _():
                issue(t + 3, 1)

            return carry

        lax.fori_loop(0, NT // 2, tbody, 0)

        pltpu.sync_copy(scores, out_hbm.at[:, pl.ds(base, BPW)])

    return body(ctr_ids, ctx_ids, neg_flat, u_global, v_global, u_reg, v_reg)


def _tc_loss(scores):
    def body(s_ref, o_ref):
        s = s_ref[...]

        def logsig(x):
            return jnp.minimum(x, 0.0) - jnp.log1p(jnp.exp(-jnp.abs(x)))

        p = logsig(jnp.clip(s[0:1, :], -10.0, 10.0))
        n = logsig(-jnp.clip(s[1:1 + NEG, :], -10.0, 10.0))
        o_ref[0, 0] = -(jnp.sum(p) + jnp.sum(n)) / BATCH

    return pl.pallas_call(
        body,
        out_shape=jax.ShapeDtypeStruct((1, 1), jnp.float32),
        out_specs=pl.BlockSpec(memory_space=pltpu.SMEM),
    )(scores)


def kernel(center_ids, context_ids, neg_ids, u_global, v_global, u_reg, v_reg):
    ctr = center_ids.astype(jnp.int32)
    ctx = context_ids.astype(jnp.int32)
    neg = neg_ids.astype(jnp.int32).reshape(-1)
    scores = _sc_scores(ctr, ctx, neg, u_global, v_global, u_reg, v_reg)
    return _tc_loss(scores)[0, 0]
